# baseline (speedup 1.0000x reference)
import functools

import jax
import jax.numpy as jnp
from jax import lax
from jax.experimental import pallas as pl
from jax.experimental.pallas import tpu as pltpu
from jax.experimental.pallas import tpu_sc as plsc

_NC = 2
_NS = 16
_L = 16
_GP = 128


def _scatter_overwrite(mem, idx, val):
    M, D = mem.shape
    B, = idx.shape
    NW = _NC * _NS
    R = M // NW
    assert M % NW == 0 and B % _L == 0

    mesh = plsc.VectorSubcoreMesh(
        core_axis_name="c", subcore_axis_name="s",
        num_cores=_NC, num_subcores=_NS)

    def body(mem_hbm, idx_hbm, val_hbm, out_hbm,
             idx_v, w_v, rowf, posf, sidb, pidb, stag,
             copy_sem, g_sem, s_sem):
        wid = lax.axis_index("s") * _NC + lax.axis_index("c")
        base = wid * R

        cdesc = pltpu.make_async_copy(
            mem_hbm.at[pl.ds(base, R)], out_hbm.at[pl.ds(base, R)], copy_sem)
        cdesc.start()

        pltpu.sync_copy(idx_hbm, idx_v)

        lanes = lax.iota(jnp.int32, _L)

        def wbuild(i, c):
            v = idx_v[pl.ds(i * _L, _L)]
            pos = i * _L + lanes
            local = v - base
            inb = (local >= 0) & (local < R)
            localc = jnp.clip(local, 0, R - 1)
            plsc.store_scatter(w_v, [localc], pos, mask=inb)
            return c
        lax.fori_loop(0, B // _L, wbuild, jnp.int32(0), unroll=False)

        def comp(i, n):
            v = idx_v[pl.ds(i * _L, _L)]
            pos = i * _L + lanes
            local = v - base
            inb = (local >= 0) & (local < R)
            localc = jnp.clip(local, 0, R - 1)
            wv = plsc.load_gather(w_v, [localc], mask=inb)
            win = inb & (wv == pos)
            plsc.store_compressed(rowf.at[pl.ds(n, _L)], v, mask=win)
            plsc.store_compressed(posf.at[pl.ds(n, _L)], pos, mask=win)
            return n + jnp.max(plsc.all_reduce_population_count(win))
        n = lax.fori_loop(0, B // _L, comp, jnp.int32(0), unroll=False)

        total = ((n + _GP - 1) // _GP) * _GP

        @pl.when(n > 0)
        def _pad():
            v0 = rowf[pl.ds(0, _L)]
            m0 = lanes < jnp.minimum(n, _L)
            pad_row = jnp.max(v0, where=m0, initial=jnp.int32(0))
            prow = jnp.full((_L,), pad_row, jnp.int32)
            ppos = plsc.load_gather(w_v, [prow - base])

            def fill(j, c):
                sl = pl.ds(j * _L, _L)
                keep = (j * _L + lanes) < n
                rowf[sl] = jnp.where(keep, rowf[sl], prow)
                posf[sl] = jnp.where(keep, posf[sl], ppos)
                return c
            lax.fori_loop(n // _L, total // _L, fill, jnp.int32(0),
                          unroll=False)

        cdesc.wait()

        def group(i, c):
            for j in range(_GP // _L):
                sl = pl.ds(j * _L, _L)
                sidb[sl] = rowf[pl.ds(i * _GP + j * _L, _L)]
                pidb[sl] = posf[pl.ds(i * _GP + j * _L, _L)]
            pltpu.async_copy(val_hbm.at[pidb], stag, g_sem).wait()
            pltpu.async_copy(stag, out_hbm.at[sidb], s_sem).wait()
            return c
        lax.fori_loop(0, total // _GP, group, jnp.int32(0), unroll=False)

    run = pl.kernel(
        body,
        out_type=jax.ShapeDtypeStruct((M, D), jnp.float32),
        mesh=mesh,
        compiler_params=pltpu.CompilerParams(
            needs_layout_passes=False, use_tc_tiling_on_sc=False),
        scratch_types=[
            pltpu.VMEM((B,), jnp.int32),
            pltpu.VMEM((R,), jnp.int32),
            pltpu.VMEM((B + _L,), jnp.int32),
            pltpu.VMEM((B + _L,), jnp.int32),
            pltpu.VMEM((_GP,), jnp.int32),
            pltpu.VMEM((_GP,), jnp.int32),
            pltpu.VMEM((_GP, D), jnp.float32),
            pltpu.SemaphoreType.DMA,
            pltpu.SemaphoreType.DMA,
            pltpu.SemaphoreType.DMA,
        ],
    )
    return run(mem, idx, val)


def kernel(mem, idx, val):
    return _scatter_overwrite(mem, idx, val)


# speedup vs baseline: 6.1420x; 6.1420x over previous
import functools

import jax
import jax.numpy as jnp
from jax import lax
from jax.experimental import pallas as pl
from jax.experimental.pallas import tpu as pltpu
from jax.experimental.pallas import tpu_sc as plsc

_NC = 2
_NS = 16
_L = 16
_GP = 128
_C = 256


def _scatter_overwrite(mem, idx, val):
    M, D = mem.shape
    B, = idx.shape
    NW = _NC * _NS
    R = M // NW
    assert M % NW == 0 and B % _L == 0

    mesh = plsc.VectorSubcoreMesh(
        core_axis_name="c", subcore_axis_name="s",
        num_cores=_NC, num_subcores=_NS)

    def body(mem_hbm, idx_hbm, val_hbm, out_hbm,
             idx_v, w_v, rowf, posf, sidb, pidb, stag, buf0, buf1,
             i_sem0, i_sem1, o_sem0, o_sem1, g_sem, s_sem):
        wid = lax.axis_index("s") * _NC + lax.axis_index("c")
        base = wid * R

        pltpu.sync_copy(idx_hbm, idx_v)

        lanes = lax.iota(jnp.int32, _L)

        def wbuild(i, c):
            v = idx_v[pl.ds(i * _L, _L)]
            pos = i * _L + lanes
            local = v - base
            inb = (local >= 0) & (local < R)
            localc = jnp.clip(local, 0, R - 1)
            plsc.store_scatter(w_v, [localc], pos, mask=inb)
            return c
        lax.fori_loop(0, B // _L, wbuild, jnp.int32(0), unroll=False)

        def comp(i, n):
            v = idx_v[pl.ds(i * _L, _L)]
            pos = i * _L + lanes
            local = v - base
            inb = (local >= 0) & (local < R)
            localc = jnp.clip(local, 0, R - 1)
            wv = plsc.load_gather(w_v, [localc], mask=inb)
            win = inb & (wv == pos)
            plsc.store_compressed(rowf.at[pl.ds(n, _L)], v, mask=win)
            plsc.store_compressed(posf.at[pl.ds(n, _L)], pos, mask=win)
            return n + jnp.max(plsc.all_reduce_population_count(win))
        n = lax.fori_loop(0, B // _L, comp, jnp.int32(0), unroll=False)

        total = ((n + _GP - 1) // _GP) * _GP

        @pl.when(n > 0)
        def _pad():
            v0 = rowf[pl.ds(0, _L)]
            m0 = lanes < jnp.minimum(n, _L)
            pad_row = jnp.max(v0, where=m0, initial=jnp.int32(0))
            prow = jnp.full((_L,), pad_row, jnp.int32)
            ppos = plsc.load_gather(w_v, [prow - base])

            def fill(j, c):
                sl = pl.ds(j * _L, _L)
                keep = (j * _L + lanes) < n
                rowf[sl] = jnp.where(keep, rowf[sl], prow)
                posf[sl] = jnp.where(keep, posf[sl], ppos)
                return c
            lax.fori_loop(n // _L, total // _L, fill, jnp.int32(0),
                          unroll=False)

        NCH = R // _C

        def cbody(k, c):
            slot = k % 2
            bufs = (buf0, buf1)
            isems = (i_sem0, i_sem1)
            osems = (o_sem0, o_sem1)
            for s in range(2):
                @pl.when((slot == s) & (k >= 2))
                def _wprev():
                    pltpu.make_async_copy(
                        bufs[s], out_hbm.at[pl.ds(base, _C)], osems[s]).wait()
                @pl.when(slot == s)
                def _go():
                    row0 = base + k * _C
                    gd = pltpu.make_async_copy(
                        mem_hbm.at[pl.ds(row0, _C)], bufs[s], isems[s])
                    gd.start()
                    gd.wait()
                    pltpu.make_async_copy(
                        bufs[s], out_hbm.at[pl.ds(row0, _C)], osems[s]).start()
            return c
        lax.fori_loop(0, NCH, cbody, jnp.int32(0), unroll=False)
        pltpu.make_async_copy(
            buf0, out_hbm.at[pl.ds(base, _C)], o_sem0).wait()
        pltpu.make_async_copy(
            buf1, out_hbm.at[pl.ds(base, _C)], o_sem1).wait()


        def group(i, c):
            for j in range(_GP // _L):
                sl = pl.ds(j * _L, _L)
                sidb[sl] = rowf[pl.ds(i * _GP + j * _L, _L)]
                pidb[sl] = posf[pl.ds(i * _GP + j * _L, _L)]
            pltpu.async_copy(val_hbm.at[pidb], stag, g_sem).wait()
            pltpu.async_copy(stag, out_hbm.at[sidb], s_sem).wait()
            return c
        lax.fori_loop(0, total // _GP, group, jnp.int32(0), unroll=False)

    run = pl.kernel(
        body,
        out_type=jax.ShapeDtypeStruct((M, D), jnp.float32),
        mesh=mesh,
        compiler_params=pltpu.CompilerParams(
            needs_layout_passes=False, use_tc_tiling_on_sc=False),
        scratch_types=[
            pltpu.VMEM((B,), jnp.int32),
            pltpu.VMEM((R,), jnp.int32),
            pltpu.VMEM((B + _L,), jnp.int32),
            pltpu.VMEM((B + _L,), jnp.int32),
            pltpu.VMEM((_GP,), jnp.int32),
            pltpu.VMEM((_GP,), jnp.int32),
            pltpu.VMEM((_GP, D), jnp.float32),
            pltpu.VMEM((_C, D), jnp.float32),
            pltpu.VMEM((_C, D), jnp.float32),
            pltpu.SemaphoreType.DMA,
            pltpu.SemaphoreType.DMA,
            pltpu.SemaphoreType.DMA,
            pltpu.SemaphoreType.DMA,
            pltpu.SemaphoreType.DMA,
            pltpu.SemaphoreType.DMA,
        ],
    )
    return run(mem, idx, val)


def kernel(mem, idx, val):
    return _scatter_overwrite(mem, idx, val)


# speedup vs baseline: 7.5276x; 1.2256x over previous
import functools

import jax
import jax.numpy as jnp
from jax import lax
from jax.experimental import pallas as pl
from jax.experimental.pallas import tpu as pltpu
from jax.experimental.pallas import tpu_sc as plsc

_NC = 2
_NS = 16
_L = 16
_GP = 32
_C = 128


def _scatter_overwrite(mem, idx, val):
    M, D = mem.shape
    B, = idx.shape
    NW = _NC * _NS
    R = M // NW
    assert M % NW == 0 and B % _L == 0

    mesh = plsc.VectorSubcoreMesh(
        core_axis_name="c", subcore_axis_name="s",
        num_cores=_NC, num_subcores=_NS)

    def body(mem_hbm, idx_hbm, val_hbm, out_hbm,
             idx_v, w_v, rowf, posf, stag, buf0, buf1,
             i_sem0, i_sem1, o_sem0, o_sem1, g_sem, s_sem):
        wid = lax.axis_index("s") * _NC + lax.axis_index("c")
        base = wid * R

        pltpu.sync_copy(idx_hbm, idx_v)

        lanes = lax.iota(jnp.int32, _L)

        def wbuild(i, c):
            v = idx_v[pl.ds(i * _L, _L)]
            pos = i * _L + lanes
            local = v - base
            inb = (local >= 0) & (local < R)
            localc = jnp.clip(local, 0, R - 1)
            plsc.store_scatter(w_v, [localc], pos, mask=inb)
            return c
        lax.fori_loop(0, B // _L, wbuild, jnp.int32(0), unroll=False)

        def comp(i, n):
            v = idx_v[pl.ds(i * _L, _L)]
            pos = i * _L + lanes
            local = v - base
            inb = (local >= 0) & (local < R)
            localc = jnp.clip(local, 0, R - 1)
            wv = plsc.load_gather(w_v, [localc], mask=inb)
            win = inb & (wv == pos)
            plsc.store_compressed(rowf.at[pl.ds(n, _L)], v, mask=win)
            plsc.store_compressed(posf.at[pl.ds(n, _L)], pos, mask=win)
            return n + jnp.max(plsc.all_reduce_population_count(win))
        n = lax.fori_loop(0, B // _L, comp, jnp.int32(0), unroll=False)

        total = ((n + _GP - 1) // _GP) * _GP

        @pl.when(n > 0)
        def _pad():
            v0 = rowf[pl.ds(0, _L)]
            m0 = lanes < jnp.minimum(n, _L)
            pad_row = jnp.max(v0, where=m0, initial=jnp.int32(0))
            prow = jnp.full((_L,), pad_row, jnp.int32)
            ppos = plsc.load_gather(w_v, [prow - base])

            def fill(j, c):
                sl = pl.ds(j * _L, _L)
                keep = (j * _L + lanes) < n
                rowf[sl] = jnp.where(keep, rowf[sl], prow)
                posf[sl] = jnp.where(keep, posf[sl], ppos)
                return c
            lax.fori_loop(n // _L, total // _L, fill, jnp.int32(0),
                          unroll=False)

        NCH = R // _C

        def cbody(k, c):
            slot = k % 2
            bufs = (buf0, buf1)
            isems = (i_sem0, i_sem1)
            osems = (o_sem0, o_sem1)
            for s in range(2):
                @pl.when((slot == s) & (k >= 2))
                def _wprev():
                    pltpu.make_async_copy(
                        bufs[s], out_hbm.at[pl.ds(base, _C)], osems[s]).wait()
                @pl.when(slot == s)
                def _go():
                    row0 = base + k * _C
                    gd = pltpu.make_async_copy(
                        mem_hbm.at[pl.ds(row0, _C)], bufs[s], isems[s])
                    gd.start()
                    gd.wait()
                    pltpu.make_async_copy(
                        bufs[s], out_hbm.at[pl.ds(row0, _C)], osems[s]).start()
            return c
        lax.fori_loop(0, NCH, cbody, jnp.int32(0), unroll=False)
        pltpu.make_async_copy(
            buf0, out_hbm.at[pl.ds(base, _C)], o_sem0).wait()
        pltpu.make_async_copy(
            buf1, out_hbm.at[pl.ds(base, _C)], o_sem1).wait()


        def batch(bi, c):
            for g in range(_GP // _L):
                pv = posf[pl.ds(bi * _GP + g * _L, _L)]
                for j in range(_L):
                    pltpu.make_async_copy(
                        val_hbm.at[pv[j]], stag.at[g * _L + j], g_sem).start()
            for j in range(_GP):
                pltpu.make_async_copy(
                    val_hbm.at[0], stag.at[j], g_sem).wait()
            for g in range(_GP // _L):
                rv = rowf[pl.ds(bi * _GP + g * _L, _L)]
                for j in range(_L):
                    pltpu.make_async_copy(
                        stag.at[g * _L + j], out_hbm.at[rv[j]], s_sem).start()
            for j in range(_GP):
                pltpu.make_async_copy(
                    stag.at[j], out_hbm.at[0], s_sem).wait()
            return c
        lax.fori_loop(0, total // _GP, batch, jnp.int32(0), unroll=False)

    run = pl.kernel(
        body,
        out_type=jax.ShapeDtypeStruct((M, D), jnp.float32),
        mesh=mesh,
        compiler_params=pltpu.CompilerParams(
            needs_layout_passes=False, use_tc_tiling_on_sc=True),
        scratch_types=[
            pltpu.VMEM((B,), jnp.int32),
            pltpu.VMEM((R,), jnp.int32),
            pltpu.VMEM((B + _L,), jnp.int32),
            pltpu.VMEM((B + _L,), jnp.int32),
            pltpu.VMEM((_GP, D), jnp.float32),
            pltpu.VMEM((_C, D), jnp.float32),
            pltpu.VMEM((_C, D), jnp.float32),
            pltpu.SemaphoreType.DMA,
            pltpu.SemaphoreType.DMA,
            pltpu.SemaphoreType.DMA,
            pltpu.SemaphoreType.DMA,
            pltpu.SemaphoreType.DMA,
            pltpu.SemaphoreType.DMA,
        ],
    )
    return run(mem, idx, val)


def kernel(mem, idx, val):
    return _scatter_overwrite(mem, idx, val)


# speedup vs baseline: 14.4668x; 1.9218x over previous
import functools

import jax
import jax.numpy as jnp
from jax import lax
from jax.experimental import pallas as pl
from jax.experimental.pallas import tpu as pltpu
from jax.experimental.pallas import tpu_sc as plsc

_NC = 2
_NS = 16
_L = 16
_C = 128


def _scatter_overwrite_t(mem_t, idx, val_flat):
    D, M = mem_t.shape
    B = idx.shape[0]
    NW = _NC * _NS
    R = M // NW
    assert M % NW == 0 and B % _L == 0 and R % _C == 0

    mesh = plsc.VectorSubcoreMesh(
        core_axis_name="c", subcore_axis_name="s",
        num_cores=_NC, num_subcores=_NS)

    def body(mem_hbm, idx_hbm, val_hbm, out_hbm,
             idx_v, w_v, buf0, buf1, colv,
             i_sem0, i_sem1, o_sem0, o_sem1, p_sem):
        wid = lax.axis_index("s") * _NC + lax.axis_index("c")
        base = wid * R

        pltpu.sync_copy(idx_hbm, idx_v)

        lanes = lax.iota(jnp.int32, _L)

        neg1 = jnp.full((_L,), -1, jnp.int32)

        def winit(i, c):
            w_v[pl.ds(i * _L, _L)] = neg1
            return c
        lax.fori_loop(0, R // _L, winit, jnp.int32(0), unroll=8)

        def wbuild(i, c):
            v = idx_v[pl.ds(i * _L, _L)]
            pos = i * _L + lanes
            local = v - base
            inb = (local >= 0) & (local < R)
            localc = jnp.clip(local, 0, R - 1)
            plsc.store_scatter(w_v, [localc], pos, mask=inb)
            return c
        lax.fori_loop(0, B // _L, wbuild, jnp.int32(0), unroll=False)

        NCH = R // _C

        def patch(k, buf):
            cb = k * _C

            def fire(vi, cnt):
                wv = w_v[pl.ds(cb + vi * _L, _L)]
                pc = plsc.all_reduce_population_count(wv >= 0)[0]

                @pl.when(pc > 0)
                def _vreg():
                    for ln in range(_L):
                        p = wv[ln]

                        @pl.when(p >= 0)
                        def _fetch(p=p, ln=ln):
                            pltpu.make_async_copy(
                                val_hbm.at[pl.ds(p * D, D)],
                                colv.at[pl.ds((vi * _L + ln) * D, D)],
                                p_sem).start()
                return cnt + pc
            cnt = lax.fori_loop(0, _C // _L, fire, jnp.int32(0),
                                unroll=False)

            @pl.when(cnt > 0)
            def _apply():
                def drain(j, c):
                    pltpu.make_async_copy(
                        val_hbm.at[pl.ds(0, D)], colv.at[pl.ds(0, D)],
                        p_sem).wait()
                    return c
                lax.fori_loop(0, cnt, drain, jnp.int32(0), unroll=False)

                def write(vi, c):
                    wv = w_v[pl.ds(cb + vi * _L, _L)]
                    pc = plsc.all_reduce_population_count(wv >= 0)[0]

                    @pl.when(pc > 0)
                    def _vreg():
                        for ln in range(_L):
                            @pl.when(wv[ln] >= 0)
                            def _write(ln=ln):
                                col = jnp.full((_L,), vi * _L + ln,
                                               jnp.int32)
                                for j in range(D // _L):
                                    v = colv[pl.ds((vi * _L + ln) * D
                                                   + j * _L, _L)]
                                    plsc.store_scatter(
                                        buf, [j * _L + lanes, col], v)
                    return c
                lax.fori_loop(0, _C // _L, write, jnp.int32(0),
                              unroll=False)

        def cbody(k, c):
            slot = k % 2
            bufs = (buf0, buf1)
            isems = (i_sem0, i_sem1)
            osems = (o_sem0, o_sem1)
            for s in range(2):
                @pl.when((slot == s) & (k >= 2))
                def _wprev():
                    pltpu.make_async_copy(
                        bufs[s], out_hbm.at[:, pl.ds(base, _C)],
                        osems[s]).wait()
                @pl.when(slot == s)
                def _go():
                    col0 = base + k * _C
                    gd = pltpu.make_async_copy(
                        mem_hbm.at[:, pl.ds(col0, _C)], bufs[s], isems[s])
                    gd.start()
                    gd.wait()
                    patch(k, bufs[s])
                    pltpu.make_async_copy(
                        bufs[s], out_hbm.at[:, pl.ds(col0, _C)],
                        osems[s]).start()
            return c
        lax.fori_loop(0, NCH, cbody, jnp.int32(0), unroll=False)
        pltpu.make_async_copy(
            buf0, out_hbm.at[:, pl.ds(base, _C)], o_sem0).wait()
        pltpu.make_async_copy(
            buf1, out_hbm.at[:, pl.ds(base, _C)], o_sem1).wait()

    run = pl.kernel(
        body,
        out_type=jax.ShapeDtypeStruct((D, M), jnp.float32),
        mesh=mesh,
        compiler_params=pltpu.CompilerParams(
            needs_layout_passes=False, use_tc_tiling_on_sc=True),
        scratch_types=[
            pltpu.VMEM((B,), jnp.int32),
            pltpu.VMEM((R,), jnp.int32),
            pltpu.VMEM((D, _C), jnp.float32),
            pltpu.VMEM((D, _C), jnp.float32),
            pltpu.VMEM((_C * D,), jnp.float32),
            pltpu.SemaphoreType.DMA,
            pltpu.SemaphoreType.DMA,
            pltpu.SemaphoreType.DMA,
            pltpu.SemaphoreType.DMA,
            pltpu.SemaphoreType.DMA,
        ],
    )
    return run(mem_t, idx, val_flat)


def kernel(mem, idx, val):
    mem_t = jnp.swapaxes(mem, 0, 1)
    val_flat = jnp.reshape(val, (-1,))
    out_t = _scatter_overwrite_t(mem_t, idx, val_flat)
    return jnp.swapaxes(out_t, 0, 1)


# speedup vs baseline: 20.5416x; 1.4199x over previous
import functools

import jax
import jax.numpy as jnp
from jax import lax
from jax.experimental import pallas as pl
from jax.experimental.pallas import tpu as pltpu
from jax.experimental.pallas import tpu_sc as plsc

_NC = 2
_NS = 16
_L = 16
_C = 256


def _scatter_overwrite_t(mem_t, idx, val_flat):
    D, M = mem_t.shape
    B = idx.shape[0]
    NW = _NC * _NS
    R = M // NW
    assert M % NW == 0 and B % _L == 0 and R % _C == 0

    mesh = plsc.VectorSubcoreMesh(
        core_axis_name="c", subcore_axis_name="s",
        num_cores=_NC, num_subcores=_NS)

    def body(mem_hbm, idx_hbm, val_hbm, out_hbm,
             idx_v, w_v, buf0, buf1, colv,
             i_sem0, i_sem1, o_sem0, o_sem1, p_sem):
        wid = lax.axis_index("s") * _NC + lax.axis_index("c")
        base = wid * R

        pltpu.sync_copy(idx_hbm, idx_v)

        lanes = lax.iota(jnp.int32, _L)

        neg1 = jnp.full((_L,), -1, jnp.int32)

        def winit(i, c):
            w_v[pl.ds(i * _L, _L)] = neg1
            return c
        lax.fori_loop(0, R // _L, winit, jnp.int32(0), unroll=8)

        def wbuild(i, c):
            v = idx_v[pl.ds(i * _L, _L)]
            pos = i * _L + lanes
            local = v - base
            inb = (local >= 0) & (local < R)
            localc = jnp.clip(local, 0, R - 1)
            plsc.store_scatter(w_v, [localc], pos, mask=inb)
            return c
        lax.fori_loop(0, B // _L, wbuild, jnp.int32(0), unroll=False)

        NCH = R // _C

        def patch_fire(k):
            cb = k * _C

            def fire(vi, cnt):
                wv = w_v[pl.ds(cb + vi * _L, _L)]
                pc = plsc.all_reduce_population_count(wv >= 0)[0]

                @pl.when(pc > 0)
                def _vreg():
                    for ln in range(_L):
                        p = wv[ln]

                        @pl.when(p >= 0)
                        def _fetch(p=p, ln=ln):
                            pltpu.make_async_copy(
                                val_hbm.at[pl.ds(p * D, D)],
                                colv.at[pl.ds((vi * _L + ln) * D, D)],
                                p_sem).start()
                return cnt + pc
            return lax.fori_loop(0, _C // _L, fire, jnp.int32(0),
                                 unroll=False)

        def patch_apply(k, buf, cnt):
            cb = k * _C

            @pl.when(cnt > 0)
            def _apply():
                def drain(j, c):
                    pltpu.make_async_copy(
                        val_hbm.at[pl.ds(0, D)], colv.at[pl.ds(0, D)],
                        p_sem).wait()
                    return c
                lax.fori_loop(0, cnt, drain, jnp.int32(0), unroll=False)

                def write(vi, c):
                    wv = w_v[pl.ds(cb + vi * _L, _L)]
                    pc = plsc.all_reduce_population_count(wv >= 0)[0]

                    @pl.when(pc > 0)
                    def _vreg():
                        for ln in range(_L):
                            @pl.when(wv[ln] >= 0)
                            def _write(ln=ln):
                                col = jnp.full((_L,), vi * _L + ln,
                                               jnp.int32)
                                for j in range(D // _L):
                                    v = colv[pl.ds((vi * _L + ln) * D
                                                   + j * _L, _L)]
                                    plsc.store_scatter(
                                        buf, [j * _L + lanes, col], v)
                    return c
                lax.fori_loop(0, _C // _L, write, jnp.int32(0),
                              unroll=False)

        bufs = (buf0, buf1)
        isems = (i_sem0, i_sem1)
        osems = (o_sem0, o_sem1)
        pltpu.make_async_copy(
            mem_hbm.at[:, pl.ds(base, _C)], buf0, i_sem0).start()

        def cbody(k, c):
            slot = k % 2
            for s in range(2):
                @pl.when(slot == s)
                def _go(s=s):
                    o = 1 - s
                    @pl.when(k >= 1)
                    def _wprev():
                        pltpu.make_async_copy(
                            bufs[o], out_hbm.at[:, pl.ds(base, _C)],
                            osems[o]).wait()
                    @pl.when(k + 1 < NCH)
                    def _pref():
                        pltpu.make_async_copy(
                            mem_hbm.at[:, pl.ds(base + (k + 1) * _C, _C)],
                            bufs[o], isems[o]).start()
                    cnt = patch_fire(k)
                    pltpu.make_async_copy(
                        mem_hbm.at[:, pl.ds(base, _C)], bufs[s],
                        isems[s]).wait()
                    patch_apply(k, bufs[s], cnt)
                    pltpu.make_async_copy(
                        bufs[s], out_hbm.at[:, pl.ds(base + k * _C, _C)],
                        osems[s]).start()
            return c
        lax.fori_loop(0, NCH, cbody, jnp.int32(0), unroll=False)
        pltpu.make_async_copy(
            bufs[(NCH - 1) % 2], out_hbm.at[:, pl.ds(base, _C)],
            osems[(NCH - 1) % 2]).wait()

    run = pl.kernel(
        body,
        out_type=jax.ShapeDtypeStruct((D, M), jnp.float32),
        mesh=mesh,
        compiler_params=pltpu.CompilerParams(
            needs_layout_passes=False, use_tc_tiling_on_sc=True),
        scratch_types=[
            pltpu.VMEM((B,), jnp.int32),
            pltpu.VMEM((R,), jnp.int32),
            pltpu.VMEM((D, _C), jnp.float32),
            pltpu.VMEM((D, _C), jnp.float32),
            pltpu.VMEM((_C * D,), jnp.float32),
            pltpu.SemaphoreType.DMA,
            pltpu.SemaphoreType.DMA,
            pltpu.SemaphoreType.DMA,
            pltpu.SemaphoreType.DMA,
            pltpu.SemaphoreType.DMA,
        ],
    )
    return run(mem_t, idx, val_flat)


def kernel(mem, idx, val):
    mem_t = jnp.swapaxes(mem, 0, 1)
    val_flat = jnp.reshape(val, (-1,))
    out_t = _scatter_overwrite_t(mem_t, idx, val_flat)
    return jnp.swapaxes(out_t, 0, 1)


# speedup vs baseline: 21.1003x; 1.0272x over previous
import functools

import jax
import jax.numpy as jnp
from jax import lax
from jax.experimental import pallas as pl
from jax.experimental.pallas import tpu as pltpu
from jax.experimental.pallas import tpu_sc as plsc

_NC = 2
_NS = 16
_L = 16
_C = 512


def _scatter_overwrite_t(mem_t, idx, val_flat):
    D, M = mem_t.shape
    B = idx.shape[0]
    NW = _NC * _NS
    R = M // NW
    assert M % NW == 0 and B % _L == 0 and (R // 2) % _C == 0

    mesh = plsc.VectorSubcoreMesh(
        core_axis_name="c", subcore_axis_name="s",
        num_cores=_NC, num_subcores=_NS)

    HR = R // 2
    SEG = 2048

    def body(mem_hbm, idx_hbm, val_hbm, out_hbm,
             idxb, w_v, buf0, buf1, colv,
             i_sem0, i_sem1, o_sem0, o_sem1, p_sem):
        wid = lax.axis_index("s") * _NC + lax.axis_index("c")
        base = wid * R

        lanes = lax.iota(jnp.int32, _L)
        neg1 = jnp.full((_L,), -1, jnp.int32)
        NCH = HR // _C

        def patch_fire(k):
            cb = k * _C

            def fire(vi, cnt):
                wv = w_v[pl.ds(cb + vi * _L, _L)]
                pc = plsc.all_reduce_population_count(wv >= 0)[0]

                @pl.when(pc > 0)
                def _vreg():
                    for ln in range(_L):
                        p = wv[ln]

                        @pl.when(p >= 0)
                        def _fetch(p=p, ln=ln):
                            pltpu.make_async_copy(
                                val_hbm.at[pl.ds(p * D, D)],
                                colv.at[pl.ds((vi * _L + ln) * D, D)],
                                p_sem).start()
                return cnt + pc
            return lax.fori_loop(0, _C // _L, fire, jnp.int32(0),
                                 unroll=False)

        def patch_apply(k, buf, cnt):
            cb = k * _C

            @pl.when(cnt > 0)
            def _apply():
                def drain(j, c):
                    pltpu.make_async_copy(
                        val_hbm.at[pl.ds(0, D)], colv.at[pl.ds(0, D)],
                        p_sem).wait()
                    return c
                lax.fori_loop(0, cnt, drain, jnp.int32(0), unroll=False)

                def write(vi, c):
                    wv = w_v[pl.ds(cb + vi * _L, _L)]
                    pc = plsc.all_reduce_population_count(wv >= 0)[0]

                    @pl.when(pc > 0)
                    def _vreg():
                        for ln in range(_L):
                            @pl.when(wv[ln] >= 0)
                            def _write(ln=ln):
                                col = jnp.full((_L,), vi * _L + ln,
                                               jnp.int32)
                                for j in range(D // _L):
                                    v = colv[pl.ds((vi * _L + ln) * D
                                                   + j * _L, _L)]
                                    plsc.store_scatter(
                                        buf, [j * _L + lanes, col], v)
                    return c
                lax.fori_loop(0, _C // _L, write, jnp.int32(0),
                              unroll=False)

        bufs = (buf0, buf1)
        isems = (i_sem0, i_sem1)
        osems = (o_sem0, o_sem1)

        def half(h, _):
            baseh = base + h * HR

            def winit(i, c):
                w_v[pl.ds(i * _L, _L)] = neg1
                return c
            lax.fori_loop(0, HR // _L, winit, jnp.int32(0), unroll=8)

            def wseg(g, c):
                pltpu.sync_copy(idx_hbm.at[pl.ds(g * SEG, SEG)], idxb)

                def wbuild(i, cc):
                    v = idxb[pl.ds(i * _L, _L)]
                    pos = g * SEG + i * _L + lanes
                    local = v - baseh
                    inb = (local >= 0) & (local < HR)
                    localc = jnp.clip(local, 0, HR - 1)
                    plsc.store_scatter(w_v, [localc], pos, mask=inb)
                    return cc
                lax.fori_loop(0, SEG // _L, wbuild, jnp.int32(0),
                              unroll=False)
                return c
            lax.fori_loop(0, B // SEG, wseg, jnp.int32(0), unroll=False)

            pltpu.make_async_copy(
                mem_hbm.at[:, pl.ds(baseh, _C)], buf0, i_sem0).start()

            def cbody(k, c):
                slot = k % 2
                for s in range(2):
                    @pl.when(slot == s)
                    def _go(s=s):
                        o = 1 - s
                        @pl.when(k >= 1)
                        def _wprev():
                            pltpu.make_async_copy(
                                bufs[o], out_hbm.at[:, pl.ds(base, _C)],
                                osems[o]).wait()
                        @pl.when(k + 1 < NCH)
                        def _pref():
                            pltpu.make_async_copy(
                                mem_hbm.at[:, pl.ds(
                                    baseh + (k + 1) * _C, _C)],
                                bufs[o], isems[o]).start()
                        cnt = patch_fire(k)
                        pltpu.make_async_copy(
                            mem_hbm.at[:, pl.ds(base, _C)], bufs[s],
                            isems[s]).wait()
                        patch_apply(k, bufs[s], cnt)
                        pltpu.make_async_copy(
                            bufs[s], out_hbm.at[:, pl.ds(
                                baseh + k * _C, _C)], osems[s]).start()
                return c
            lax.fori_loop(0, NCH, cbody, jnp.int32(0), unroll=False)
            pltpu.make_async_copy(
                bufs[(NCH - 1) % 2], out_hbm.at[:, pl.ds(base, _C)],
                osems[(NCH - 1) % 2]).wait()
            return _
        lax.fori_loop(0, 2, half, jnp.int32(0), unroll=False)

    run = pl.kernel(
        body,
        out_type=jax.ShapeDtypeStruct((D, M), jnp.float32),
        mesh=mesh,
        compiler_params=pltpu.CompilerParams(
            needs_layout_passes=False, use_tc_tiling_on_sc=True),
        scratch_types=[
            pltpu.VMEM((SEG,), jnp.int32),
            pltpu.VMEM((HR,), jnp.int32),
            pltpu.VMEM((D, _C), jnp.float32),
            pltpu.VMEM((D, _C), jnp.float32),
            pltpu.VMEM((_C * D,), jnp.float32),
            pltpu.SemaphoreType.DMA,
            pltpu.SemaphoreType.DMA,
            pltpu.SemaphoreType.DMA,
            pltpu.SemaphoreType.DMA,
            pltpu.SemaphoreType.DMA,
        ],
    )
    return run(mem_t, idx, val_flat)


def kernel(mem, idx, val):
    mem_t = jnp.swapaxes(mem, 0, 1)
    val_flat = jnp.reshape(val, (-1,))
    out_t = _scatter_overwrite_t(mem_t, idx, val_flat)
    return jnp.swapaxes(out_t, 0, 1)


# speedup vs baseline: 22.2920x; 1.0565x over previous
import functools

import jax
import jax.numpy as jnp
from jax import lax
from jax.experimental import pallas as pl
from jax.experimental.pallas import tpu as pltpu
from jax.experimental.pallas import tpu_sc as plsc

_NC = 2
_NS = 16
_L = 16
_C = 512


def _scatter_overwrite_t(mem_t, idx, val_flat):
    D, M = mem_t.shape
    B = idx.shape[0]
    NW = _NC * _NS
    R = M // NW
    assert M % NW == 0 and B % _L == 0 and (R // 2) % _C == 0

    mesh = plsc.VectorSubcoreMesh(
        core_axis_name="c", subcore_axis_name="s",
        num_cores=_NC, num_subcores=_NS)

    HR = R // 2
    SEG = 2048

    def body(mem_hbm, idx_hbm, val_hbm, out_hbm,
             idxb, w_v, buf0, buf1, colv,
             i_sem0, i_sem1, o_sem0, o_sem1, p_sem):
        wid = lax.axis_index("s") * _NC + lax.axis_index("c")
        base = wid * R

        lanes = lax.iota(jnp.int32, _L)
        neg1 = jnp.full((_L,), -1, jnp.int32)
        NCH = HR // _C

        def patch_fire(k, enable):
            cb = k * _C

            def fire(vi, cnt):
                wv = w_v[pl.ds(cb + vi * _L, _L)]
                pc = plsc.all_reduce_population_count(wv >= 0)[0]

                @pl.when(enable & (pc > 0))
                def _vreg():
                    for ln in range(_L):
                        p = wv[ln]

                        @pl.when(p >= 0)
                        def _fetch(p=p, ln=ln):
                            pltpu.make_async_copy(
                                val_hbm.at[pl.ds(p * D, D)],
                                colv.at[pl.ds((vi * _L + ln) * D, D)],
                                p_sem).start()
                return cnt + jnp.where(enable, pc, 0)
            return lax.fori_loop(0, _C // _L, fire, jnp.int32(0),
                                 unroll=False)

        def patch_apply(k, buf, cnt):
            cb = k * _C

            @pl.when(cnt > 0)
            def _apply():
                def drain(j, c):
                    pltpu.make_async_copy(
                        val_hbm.at[pl.ds(0, D)], colv.at[pl.ds(0, D)],
                        p_sem).wait()
                    return c
                lax.fori_loop(0, cnt, drain, jnp.int32(0), unroll=False)

                def write(vi, c):
                    wv = w_v[pl.ds(cb + vi * _L, _L)]
                    pc = plsc.all_reduce_population_count(wv >= 0)[0]

                    @pl.when(pc > 0)
                    def _vreg():
                        for ln in range(_L):
                            @pl.when(wv[ln] >= 0)
                            def _write(ln=ln):
                                col = jnp.full((_L,), vi * _L + ln,
                                               jnp.int32)
                                for j in range(D // _L):
                                    v = colv[pl.ds((vi * _L + ln) * D
                                                   + j * _L, _L)]
                                    plsc.store_scatter(
                                        buf, [j * _L + lanes, col], v)
                    return c
                lax.fori_loop(0, _C // _L, write, jnp.int32(0),
                              unroll=False)

        bufs = (buf0, buf1)
        isems = (i_sem0, i_sem1)
        osems = (o_sem0, o_sem1)

        def half(h, _):
            baseh = base + h * HR

            def winit(i, c):
                w_v[pl.ds(i * _L, _L)] = neg1
                return c
            lax.fori_loop(0, HR // _L, winit, jnp.int32(0), unroll=8)

            def wseg(g, c):
                pltpu.sync_copy(idx_hbm.at[pl.ds(g * SEG, SEG)], idxb)

                def wbuild(i, cc):
                    v = idxb[pl.ds(i * _L, _L)]
                    pos = g * SEG + i * _L + lanes
                    local = v - baseh
                    inb = (local >= 0) & (local < HR)
                    localc = jnp.clip(local, 0, HR - 1)
                    plsc.store_scatter(w_v, [localc], pos, mask=inb)
                    return cc
                lax.fori_loop(0, SEG // _L, wbuild, jnp.int32(0),
                              unroll=False)
                return c
            lax.fori_loop(0, B // SEG, wseg, jnp.int32(0), unroll=False)

            pltpu.make_async_copy(
                mem_hbm.at[:, pl.ds(baseh, _C)], buf0, i_sem0).start()
            cnt0 = patch_fire(0, jnp.bool_(True))

            def cbody(k, cnt_cur):
                slot = k % 2
                for s in range(2):
                    @pl.when(slot == s)
                    def _go(s=s):
                        o = 1 - s
                        @pl.when(k >= 1)
                        def _wprev():
                            pltpu.make_async_copy(
                                bufs[o], out_hbm.at[:, pl.ds(base, _C)],
                                osems[o]).wait()
                        @pl.when(k + 1 < NCH)
                        def _pref():
                            pltpu.make_async_copy(
                                mem_hbm.at[:, pl.ds(
                                    baseh + (k + 1) * _C, _C)],
                                bufs[o], isems[o]).start()
                        pltpu.make_async_copy(
                            mem_hbm.at[:, pl.ds(base, _C)], bufs[s],
                            isems[s]).wait()
                        patch_apply(k, bufs[s], cnt_cur)
                kk = jnp.minimum(k + 1, NCH - 1)
                cnt_next = patch_fire(kk, k + 1 < NCH)
                for s in range(2):
                    @pl.when(slot == s)
                    def _sc(s=s):
                        pltpu.make_async_copy(
                            bufs[s], out_hbm.at[:, pl.ds(
                                baseh + k * _C, _C)], osems[s]).start()
                return cnt_next
            lax.fori_loop(0, NCH, cbody, cnt0, unroll=False)
            pltpu.make_async_copy(
                bufs[(NCH - 1) % 2], out_hbm.at[:, pl.ds(base, _C)],
                osems[(NCH - 1) % 2]).wait()
            return _
        lax.fori_loop(0, 2, half, jnp.int32(0), unroll=False)

    run = pl.kernel(
        body,
        out_type=jax.ShapeDtypeStruct((D, M), jnp.float32),
        mesh=mesh,
        compiler_params=pltpu.CompilerParams(
            needs_layout_passes=False, use_tc_tiling_on_sc=True),
        scratch_types=[
            pltpu.VMEM((SEG,), jnp.int32),
            pltpu.VMEM((HR,), jnp.int32),
            pltpu.VMEM((D, _C), jnp.float32),
            pltpu.VMEM((D, _C), jnp.float32),
            pltpu.VMEM((_C * D,), jnp.float32),
            pltpu.SemaphoreType.DMA,
            pltpu.SemaphoreType.DMA,
            pltpu.SemaphoreType.DMA,
            pltpu.SemaphoreType.DMA,
            pltpu.SemaphoreType.DMA,
        ],
    )
    return run(mem_t, idx, val_flat)


def kernel(mem, idx, val):
    mem_t = jnp.swapaxes(mem, 0, 1)
    val_flat = jnp.reshape(val, (-1,))
    out_t = _scatter_overwrite_t(mem_t, idx, val_flat)
    return jnp.swapaxes(out_t, 0, 1)


# speedup vs baseline: 23.9932x; 1.0763x over previous
import functools

import jax
import jax.numpy as jnp
from jax import lax
from jax.experimental import pallas as pl
from jax.experimental.pallas import tpu as pltpu
from jax.experimental.pallas import tpu_sc as plsc

_NC = 2
_NS = 16
_L = 16
_C = 256


def _scatter_overwrite_t(mem_t, idx, val_flat):
    D, M = mem_t.shape
    B = idx.shape[0]
    NW = _NC * _NS
    R = M // NW
    assert M % NW == 0 and B % _L == 0 and (R // 2) % _C == 0

    mesh = plsc.VectorSubcoreMesh(
        core_axis_name="c", subcore_axis_name="s",
        num_cores=_NC, num_subcores=_NS)

    HR = R // 2
    SEG = 2048

    def body(mem_hbm, idx_hbm, val_hbm, out_hbm,
             idxb, w_v, buf0, buf1, buf2, buf3, colv,
             i_sem0, i_sem1, i_sem2, i_sem3,
             o_sem0, o_sem1, o_sem2, o_sem3, p_sem):
        wid = lax.axis_index("s") * _NC + lax.axis_index("c")
        base = wid * R

        lanes = lax.iota(jnp.int32, _L)
        neg1 = jnp.full((_L,), -1, jnp.int32)
        NCH = HR // _C

        def patch_fire(k, enable):
            cb = k * _C

            def fire(vi, cnt):
                wv = w_v[pl.ds(cb + vi * _L, _L)]
                pc = plsc.all_reduce_population_count(wv >= 0)[0]

                @pl.when(enable & (pc > 0))
                def _vreg():
                    for ln in range(_L):
                        p = wv[ln]

                        @pl.when(p >= 0)
                        def _fetch(p=p, ln=ln):
                            pltpu.make_async_copy(
                                val_hbm.at[pl.ds(p * D, D)],
                                colv.at[pl.ds((vi * _L + ln) * D, D)],
                                p_sem).start()
                return cnt + jnp.where(enable, pc, 0)
            return lax.fori_loop(0, _C // _L, fire, jnp.int32(0),
                                 unroll=False)

        def patch_apply(k, buf, cnt):
            cb = k * _C

            @pl.when(cnt > 0)
            def _apply():
                def drain(j, c):
                    pltpu.make_async_copy(
                        val_hbm.at[pl.ds(0, D)], colv.at[pl.ds(0, D)],
                        p_sem).wait()
                    return c
                lax.fori_loop(0, cnt, drain, jnp.int32(0), unroll=False)

                def write(vi, c):
                    wv = w_v[pl.ds(cb + vi * _L, _L)]
                    pc = plsc.all_reduce_population_count(wv >= 0)[0]

                    @pl.when(pc > 0)
                    def _vreg():
                        for ln in range(_L):
                            @pl.when(wv[ln] >= 0)
                            def _write(ln=ln):
                                col = jnp.full((_L,), vi * _L + ln,
                                               jnp.int32)
                                for j in range(D // _L):
                                    v = colv[pl.ds((vi * _L + ln) * D
                                                   + j * _L, _L)]
                                    plsc.store_scatter(
                                        buf, [j * _L + lanes, col], v)
                    return c
                lax.fori_loop(0, _C // _L, write, jnp.int32(0),
                              unroll=False)

        bufs = (buf0, buf1, buf2, buf3)
        isems = (i_sem0, i_sem1, i_sem2, i_sem3)
        osems = (o_sem0, o_sem1, o_sem2, o_sem3)

        def half(h, _):
            baseh = base + h * HR

            def winit(i, c):
                w_v[pl.ds(i * _L, _L)] = neg1
                return c
            lax.fori_loop(0, HR // _L, winit, jnp.int32(0), unroll=8)

            def wseg(g, c):
                pltpu.sync_copy(idx_hbm.at[pl.ds(g * SEG, SEG)], idxb)

                def wbuild(i, cc):
                    v = idxb[pl.ds(i * _L, _L)]
                    pos = g * SEG + i * _L + lanes
                    local = v - baseh
                    inb = (local >= 0) & (local < HR)
                    localc = jnp.clip(local, 0, HR - 1)
                    plsc.store_scatter(w_v, [localc], pos, mask=inb)
                    return cc
                lax.fori_loop(0, SEG // _L, wbuild, jnp.int32(0),
                              unroll=False)
                return c
            lax.fori_loop(0, B // SEG, wseg, jnp.int32(0), unroll=False)

            pltpu.make_async_copy(
                mem_hbm.at[:, pl.ds(baseh, _C)], buf0, i_sem0).start()
            pltpu.make_async_copy(
                mem_hbm.at[:, pl.ds(baseh + _C, _C)], buf1, i_sem1).start()
            cnt0 = patch_fire(0, jnp.bool_(True))

            def cbody(k, cnt_cur):
                slot = k % 4
                for s in range(4):
                    @pl.when(slot == s)
                    def _go(s=s):
                        o = (s + 2) % 4
                        @pl.when(k >= 2)
                        def _wprev():
                            pltpu.make_async_copy(
                                bufs[o], out_hbm.at[:, pl.ds(base, _C)],
                                osems[o]).wait()
                        @pl.when(k + 2 < NCH)
                        def _pref():
                            pltpu.make_async_copy(
                                mem_hbm.at[:, pl.ds(
                                    baseh + (k + 2) * _C, _C)],
                                bufs[o], isems[o]).start()
                        pltpu.make_async_copy(
                            mem_hbm.at[:, pl.ds(base, _C)], bufs[s],
                            isems[s]).wait()
                        patch_apply(k, bufs[s], cnt_cur)
                        pltpu.make_async_copy(
                            bufs[s], out_hbm.at[:, pl.ds(
                                baseh + k * _C, _C)], osems[s]).start()
                kk = jnp.minimum(k + 1, NCH - 1)
                return patch_fire(kk, k + 1 < NCH)
            lax.fori_loop(0, NCH, cbody, cnt0, unroll=False)
            for j in (NCH - 2, NCH - 1):
                pltpu.make_async_copy(
                    bufs[j % 4], out_hbm.at[:, pl.ds(base, _C)],
                    osems[j % 4]).wait()
            return _
        lax.fori_loop(0, 2, half, jnp.int32(0), unroll=False)

    run = pl.kernel(
        body,
        out_type=jax.ShapeDtypeStruct((D, M), jnp.float32),
        mesh=mesh,
        compiler_params=pltpu.CompilerParams(
            needs_layout_passes=False, use_tc_tiling_on_sc=True),
        scratch_types=[
            pltpu.VMEM((SEG,), jnp.int32),
            pltpu.VMEM((HR,), jnp.int32),
            pltpu.VMEM((D, _C), jnp.float32),
            pltpu.VMEM((D, _C), jnp.float32),
            pltpu.VMEM((D, _C), jnp.float32),
            pltpu.VMEM((D, _C), jnp.float32),
            pltpu.VMEM((_C * D,), jnp.float32),
            pltpu.SemaphoreType.DMA,
            pltpu.SemaphoreType.DMA,
            pltpu.SemaphoreType.DMA,
            pltpu.SemaphoreType.DMA,
            pltpu.SemaphoreType.DMA,
            pltpu.SemaphoreType.DMA,
            pltpu.SemaphoreType.DMA,
            pltpu.SemaphoreType.DMA,
            pltpu.SemaphoreType.DMA,
        ],
    )
    return run(mem_t, idx, val_flat)


def kernel(mem, idx, val):
    mem_t = jnp.swapaxes(mem, 0, 1)
    val_flat = jnp.reshape(val, (-1,))
    out_t = _scatter_overwrite_t(mem_t, idx, val_flat)
    return jnp.swapaxes(out_t, 0, 1)


# speedup vs baseline: 24.2080x; 1.0090x over previous
import functools

import jax
import jax.numpy as jnp
from jax import lax
from jax.experimental import pallas as pl
from jax.experimental.pallas import tpu as pltpu
from jax.experimental.pallas import tpu_sc as plsc

_NC = 2
_NS = 16
_L = 16
_C = 256


def _scatter_overwrite_t(mem_t, idx, val_flat):
    D, M = mem_t.shape
    B = idx.shape[0]
    NW = _NC * _NS
    R = M // NW
    assert M % NW == 0 and B % _L == 0 and (R // 2) % _C == 0

    mesh = plsc.VectorSubcoreMesh(
        core_axis_name="c", subcore_axis_name="s",
        num_cores=_NC, num_subcores=_NS)

    HR = R // 2
    SEG = 2048

    def body(mem_hbm, idx_hbm, val_hbm, out_hbm,
             idxb, w_v, buf0, buf1, buf2, buf3, colv, msumb,
             i_sem0, i_sem1, i_sem2, i_sem3,
             o_sem0, o_sem1, o_sem2, o_sem3, p_sem):
        wid = lax.axis_index("s") * _NC + lax.axis_index("c")
        base = wid * R

        lanes = lax.iota(jnp.int32, _L)
        neg1 = jnp.full((_L,), -1, jnp.int32)
        NCH = HR // _C

        def patch_fire(k, enable):
            cb = k * _C

            def fire(vi, carry):
                cnt, msum = carry
                wv = w_v[pl.ds(cb + vi * _L, _L)]
                pc = plsc.all_reduce_population_count(wv >= 0)[0]

                @pl.when(enable & (pc > 0))
                def _vreg():
                    for ln in range(_L):
                        p = wv[ln]

                        @pl.when(p >= 0)
                        def _fetch(p=p, ln=ln):
                            pltpu.make_async_copy(
                                val_hbm.at[pl.ds(p * D, D)],
                                colv.at[pl.ds((vi * _L + ln) * D, D)],
                                p_sem).start()
                msum = jnp.where(lanes == vi, pc, msum)
                return cnt + jnp.where(enable, pc, 0), msum
            cnt, msum = lax.fori_loop(
                0, _C // _L, fire,
                (jnp.int32(0), jnp.zeros((_L,), jnp.int32)), unroll=False)
            msumb[pl.ds(0, _L)] = msum
            return cnt

        def patch_apply(k, buf, cnt):
            cb = k * _C

            @pl.when(cnt > 0)
            def _apply():
                def drain(j, c):
                    pltpu.make_async_copy(
                        val_hbm.at[pl.ds(0, D)], colv.at[pl.ds(0, D)],
                        p_sem).wait()
                    return c
                lax.fori_loop(0, cnt, drain, jnp.int32(0), unroll=False)
                msum = msumb[pl.ds(0, _L)]

                def write(vi, c):
                    mv = msum.at[jnp.full((_L,), vi, jnp.int32)].get(
                        mode="promise_in_bounds")

                    @pl.when(mv[0] > 0)
                    def _vreg():
                        wv = w_v[pl.ds(cb + vi * _L, _L)]
                        for ln in range(_L):
                            @pl.when(wv[ln] >= 0)
                            def _write(ln=ln):
                                col = jnp.full((_L,), vi * _L + ln,
                                               jnp.int32)
                                for j in range(D // _L):
                                    v = colv[pl.ds((vi * _L + ln) * D
                                                   + j * _L, _L)]
                                    plsc.store_scatter(
                                        buf, [j * _L + lanes, col], v)
                    return c
                lax.fori_loop(0, _C // _L, write, jnp.int32(0),
                              unroll=False)

        bufs = (buf0, buf1, buf2, buf3)
        isems = (i_sem0, i_sem1, i_sem2, i_sem3)
        osems = (o_sem0, o_sem1, o_sem2, o_sem3)

        def half(h, _):
            baseh = base + h * HR

            def winit(i, c):
                w_v[pl.ds(i * _L, _L)] = neg1
                return c
            lax.fori_loop(0, HR // _L, winit, jnp.int32(0), unroll=8)

            def wseg(g, c):
                pltpu.sync_copy(idx_hbm.at[pl.ds(g * SEG, SEG)], idxb)

                def wbuild(i, cc):
                    v = idxb[pl.ds(i * _L, _L)]
                    pos = g * SEG + i * _L + lanes
                    local = v - baseh
                    inb = (local >= 0) & (local < HR)
                    localc = jnp.clip(local, 0, HR - 1)
                    plsc.store_scatter(w_v, [localc], pos, mask=inb)
                    return cc
                lax.fori_loop(0, SEG // _L, wbuild, jnp.int32(0),
                              unroll=False)
                return c
            lax.fori_loop(0, B // SEG, wseg, jnp.int32(0), unroll=False)

            pltpu.make_async_copy(
                mem_hbm.at[:, pl.ds(baseh, _C)], buf0, i_sem0).start()
            pltpu.make_async_copy(
                mem_hbm.at[:, pl.ds(baseh + _C, _C)], buf1, i_sem1).start()
            cnt0 = patch_fire(0, jnp.bool_(True))

            def cbody(k, cnt_cur):
                slot = k % 4
                for s in range(4):
                    @pl.when(slot == s)
                    def _go(s=s):
                        o = (s + 2) % 4
                        @pl.when(k >= 2)
                        def _wprev():
                            pltpu.make_async_copy(
                                bufs[o], out_hbm.at[:, pl.ds(base, _C)],
                                osems[o]).wait()
                        @pl.when(k + 2 < NCH)
                        def _pref():
                            pltpu.make_async_copy(
                                mem_hbm.at[:, pl.ds(
                                    baseh + (k + 2) * _C, _C)],
                                bufs[o], isems[o]).start()
                        pltpu.make_async_copy(
                            mem_hbm.at[:, pl.ds(base, _C)], bufs[s],
                            isems[s]).wait()
                        patch_apply(k, bufs[s], cnt_cur)
                        pltpu.make_async_copy(
                            bufs[s], out_hbm.at[:, pl.ds(
                                baseh + k * _C, _C)], osems[s]).start()
                kk = jnp.minimum(k + 1, NCH - 1)
                return patch_fire(kk, k + 1 < NCH)
            lax.fori_loop(0, NCH, cbody, cnt0, unroll=False)
            for j in (NCH - 2, NCH - 1):
                pltpu.make_async_copy(
                    bufs[j % 4], out_hbm.at[:, pl.ds(base, _C)],
                    osems[j % 4]).wait()
            return _
        lax.fori_loop(0, 2, half, jnp.int32(0), unroll=False)

    run = pl.kernel(
        body,
        out_type=jax.ShapeDtypeStruct((D, M), jnp.float32),
        mesh=mesh,
        compiler_params=pltpu.CompilerParams(
            needs_layout_passes=False, use_tc_tiling_on_sc=True),
        scratch_types=[
            pltpu.VMEM((SEG,), jnp.int32),
            pltpu.VMEM((HR,), jnp.int32),
            pltpu.VMEM((D, _C), jnp.float32),
            pltpu.VMEM((D, _C), jnp.float32),
            pltpu.VMEM((D, _C), jnp.float32),
            pltpu.VMEM((D, _C), jnp.float32),
            pltpu.VMEM((_C * D,), jnp.float32),
            pltpu.VMEM((_L,), jnp.int32),
            pltpu.SemaphoreType.DMA,
            pltpu.SemaphoreType.DMA,
            pltpu.SemaphoreType.DMA,
            pltpu.SemaphoreType.DMA,
            pltpu.SemaphoreType.DMA,
            pltpu.SemaphoreType.DMA,
            pltpu.SemaphoreType.DMA,
            pltpu.SemaphoreType.DMA,
            pltpu.SemaphoreType.DMA,
        ],
    )
    return run(mem_t, idx, val_flat)


def kernel(mem, idx, val):
    mem_t = jnp.swapaxes(mem, 0, 1)
    val_flat = jnp.reshape(val, (-1,))
    out_t = _scatter_overwrite_t(mem_t, idx, val_flat)
    return jnp.swapaxes(out_t, 0, 1)
